# baseline (speedup 1.0000x reference)
import math

import jax
import jax.numpy as jnp
from jax.experimental import pallas as pl
from jax.experimental.pallas import tpu as pltpu

_N_VOCAB = 1000000
_N_DIM = 64
_SIGMA = 1.0

_LANES = 128
_ROWS_TOTAL = _N_VOCAB * _N_DIM // _LANES
_BLOCK_ROWS = 10000
_NUM_BLOCKS = _ROWS_TOTAL // _BLOCK_ROWS

_N_ELEMS = 2 * _N_VOCAB * _N_DIM
_CONST = _N_ELEMS * (-math.log(_SIGMA) - 0.5 * math.log(2.0 * math.pi))


def _prior_body(w_ref, c_ref, o_ref):
    i = pl.program_id(0)

    w = w_ref[...]
    c = c_ref[...]
    part = jnp.sum(w * w) + jnp.sum(c * c)

    @pl.when(i == 0)
    def _init():
        o_ref[0, 0] = 0.0

    o_ref[0, 0] += part

    @pl.when(i == _NUM_BLOCKS - 1)
    def _finalize():
        o_ref[0, 0] = -0.5 * o_ref[0, 0] + _CONST


def kernel(target_ixs, context_ixs, negative_sample_ixs, word_weight, context_weight):
    del target_ixs, context_ixs, negative_sample_ixs
    w = word_weight.reshape(_ROWS_TOTAL, _LANES)
    c = context_weight.reshape(_ROWS_TOTAL, _LANES)

    out = pl.pallas_call(
        _prior_body,
        grid=(_NUM_BLOCKS,),
        in_specs=[
            pl.BlockSpec((_BLOCK_ROWS, _LANES), lambda i: (i, 0)),
            pl.BlockSpec((_BLOCK_ROWS, _LANES), lambda i: (i, 0)),
        ],
        out_specs=pl.BlockSpec(
            (1, 1), lambda i: (0, 0), memory_space=pltpu.MemorySpace.SMEM
        ),
        out_shape=jax.ShapeDtypeStruct((1, 1), jnp.float32),
    )(w, c)
    return out[0, 0]


# speedup vs baseline: 1.2785x; 1.2785x over previous
import math

import jax
import jax.numpy as jnp
from jax.experimental import pallas as pl
from jax.experimental.pallas import tpu as pltpu

_N_VOCAB = 1000000
_N_DIM = 64
_SIGMA = 1.0

_LANES = _N_DIM
_ROWS_TOTAL = _N_VOCAB
_BLOCK_ROWS = 20000
_NUM_BLOCKS = _ROWS_TOTAL // _BLOCK_ROWS

_N_ELEMS = 2 * _N_VOCAB * _N_DIM
_CONST = _N_ELEMS * (-math.log(_SIGMA) - 0.5 * math.log(2.0 * math.pi))


def _prior_body(w_ref, c_ref, o_ref):
    i = pl.program_id(0)

    w = w_ref[...]
    c = c_ref[...]
    part = jnp.sum(w * w) + jnp.sum(c * c)

    @pl.when(i == 0)
    def _init():
        o_ref[0, 0] = 0.0

    o_ref[0, 0] += part

    @pl.when(i == _NUM_BLOCKS - 1)
    def _finalize():
        o_ref[0, 0] = -0.5 * o_ref[0, 0] + _CONST


def kernel(target_ixs, context_ixs, negative_sample_ixs, word_weight, context_weight):
    del target_ixs, context_ixs, negative_sample_ixs
    w = word_weight
    c = context_weight

    out = pl.pallas_call(
        _prior_body,
        grid=(_NUM_BLOCKS,),
        in_specs=[
            pl.BlockSpec((_BLOCK_ROWS, _LANES), lambda i: (i, 0)),
            pl.BlockSpec((_BLOCK_ROWS, _LANES), lambda i: (i, 0)),
        ],
        out_specs=pl.BlockSpec(
            (1, 1), lambda i: (0, 0), memory_space=pltpu.MemorySpace.SMEM
        ),
        out_shape=jax.ShapeDtypeStruct((1, 1), jnp.float32),
    )(w, c)
    return out[0, 0]
